# Initial kernel scaffold; baseline (speedup 1.0000x reference)
import jax
import jax.numpy as jnp
from jax.experimental import pallas as pl


def kernel(x, gate_w, e_bias, wg, wu, wd, sg, su, sd):
    raise NotImplementedError("write your pallas kernel here")


# speedup vs baseline: 1.9794x; 1.9794x over previous
import functools

import jax
import jax.numpy as jnp
from jax.experimental import pallas as pl

B, S, H = 1, 2048, 2048
I = 1024
E = 8
K = 2


def _routing_body(x_ref, gw_ref, bias_ref, c_ref, mv_ref):
    xf = x_ref[...]
    scores = jax.lax.dot_general(
        xf, gw_ref[...], (((1,), (1,)), ((), ())),
        preferred_element_type=jnp.float32,
    )
    biased = scores + bias_ref[...]
    iota = jax.lax.broadcasted_iota(jnp.int32, (S, E), 1)
    neg_inf = jnp.float32(-jnp.inf)

    v1 = jnp.max(biased, axis=1, keepdims=True)
    idx1 = jnp.min(jnp.where(biased == v1, iota, E), axis=1, keepdims=True)
    masked = jnp.where(iota == idx1, neg_inf, biased)
    v2 = jnp.max(masked, axis=1, keepdims=True)
    idx2 = jnp.min(jnp.where(masked == v2, iota, E), axis=1, keepdims=True)

    u1 = jnp.max(scores, axis=1, keepdims=True)
    uidx1 = jnp.min(jnp.where(scores == u1, iota, E), axis=1, keepdims=True)
    umask = jnp.where(iota == uidx1, neg_inf, scores)
    u2 = jnp.max(umask, axis=1, keepdims=True)
    p1 = jax.nn.sigmoid(u1)
    p2 = jax.nn.sigmoid(u2)
    ps = p1 + p2
    p1 = p1 / ps
    p2 = p2 / ps

    oh1 = (iota == idx1).astype(jnp.float32)
    oh2 = (iota == idx2).astype(jnp.float32)
    c_ref[...] = oh1 * p1 + oh2 * p2

    counts = jnp.sum(oh1 + oh2, axis=0, keepdims=True)
    freq = counts / jnp.float32(S * K)
    fmean = jnp.sum(freq) / jnp.float32(E)
    mv_ref[...] = jnp.full((1, 1), (jnp.max(freq) - fmean) / fmean, jnp.float32)


def _moe_body(x_ref, wg_ref, wu_ref, wd_ref, coef_ref, out_ref):
    e = pl.program_id(1)
    g = jax.lax.dot_general(
        x_ref[...], wg_ref[0], (((1,), (1,)), ((), ())),
        preferred_element_type=jnp.float32)
    u = jax.lax.dot_general(
        x_ref[...], wu_ref[0], (((1,), (1,)), ((), ())),
        preferred_element_type=jnp.float32)
    h = (jax.nn.silu(g) * u).astype(jnp.bfloat16)
    y = jax.lax.dot_general(
        h, wd_ref[0], (((1,), (1,)), ((), ())),
        preferred_element_type=jnp.float32)
    contrib = y * coef_ref[0]

    @pl.when(e == 0)
    def _init():
        out_ref[...] = contrib

    @pl.when(e > 0)
    def _acc():
        out_ref[...] += contrib


@jax.jit
def kernel(x, gate_w, e_bias, wg, wu, wd, sg, su, sd):
    xf = x.reshape(S, H)

    c, mv = pl.pallas_call(
        _routing_body,
        out_shape=(
            jax.ShapeDtypeStruct((S, E), jnp.float32),
            jax.ShapeDtypeStruct((1, 1), jnp.float32),
        ),
    )(xf, gate_w, e_bias.reshape(1, E))

    wg_all = jnp.concatenate([sg[None], wg], axis=0).astype(jnp.bfloat16)
    wu_all = jnp.concatenate([su[None], wu], axis=0).astype(jnp.bfloat16)
    wd_all = jnp.concatenate([sd[None], wd], axis=0).astype(jnp.bfloat16)
    coef = jnp.concatenate([jnp.ones((S, 1), jnp.float32), c], axis=1)
    coef_t = coef.T.reshape(E + 1, S, 1)

    x_bf = xf.astype(jnp.bfloat16)

    nt = 2
    tt = S // nt
    out = pl.pallas_call(
        _moe_body,
        grid=(nt, E + 1),
        in_specs=[
            pl.BlockSpec((tt, H), lambda t, e: (t, 0)),
            pl.BlockSpec((1, I, H), lambda t, e: (e, 0, 0)),
            pl.BlockSpec((1, I, H), lambda t, e: (e, 0, 0)),
            pl.BlockSpec((1, H, I), lambda t, e: (e, 0, 0)),
            pl.BlockSpec((1, tt, 1), lambda t, e: (e, t, 0)),
        ],
        out_specs=pl.BlockSpec((tt, H), lambda t, e: (t, 0)),
        out_shape=jax.ShapeDtypeStruct((S, H), jnp.float32),
    )(x_bf, wg_all, wu_all, wd_all, coef_t)

    return (out.reshape(B, S, H), jnp.float32(0.0), mv[0, 0])
